# baseline (speedup 1.0000x reference)
import functools

import jax
import jax.numpy as jnp
from jax import lax
from jax.experimental import pallas as pl
from jax.experimental.pallas import tpu as pltpu
from jax.experimental.pallas import tpu_sc as plsc

N = 10000
D = 128
NEG_SLOP = 0.2

NC = 2
NS = 16
L = 16
NW = NC * NS
B = 128
NPT = 640
NP = NS * NPT
RB = 1024

_MESH = plsc.VectorSubcoreMesh(
    core_axis_name="c", subcore_axis_name="s", num_cores=NC, num_subcores=NS)
_SC_PARAMS = pltpu.CompilerParams(needs_layout_passes=False)
_SPLIT_NUM, _SPLIT_DEN = 21, 32
_SPLIT0_NUM, _SPLIT0_DEN = 23, 32
_HIGH = lax.Precision.HIGHEST


def _leaky(v):
    return jnp.where(v >= 0, v, NEG_SLOP * v)


def _scale_rows(rows, coefv):
    @plsc.parallel_loop(0, B, 1, unroll=8)
    def _(i):
        cvec = plsc.load_gather(coefv, [jnp.full((L,), i, jnp.int32)])
        for j in range(D // L):
            rows[i, pl.ds(j * L, L)] = rows[i, pl.ds(j * L, L)] * cvec


def _load_set(sew, dsth, seb, dstb, setidx, p):
    o4 = pl.multiple_of(p * (4 * B), B)
    o2 = pl.multiple_of(p * (2 * B), B)
    pltpu.sync_copy(sew.at[pl.ds(o4, 4 * B)], seb.at[setidx])
    pltpu.sync_copy(dsth.at[pl.ds(o2, B)], dstb.at[setidx, 0])
    pltpu.sync_copy(dsth.at[pl.ds(o2 + B, B)], dstb.at[setidx, 1])


def _edge_pass_body(na, nb, mode, hlin, src_args, mp, dp_or_none,
                    seb, dstb, coefv, rowsa, rowsb, tblv, acc,
                    sema, semb, semsa, semsb):
    if mode == 0:
        sew, dsth, znd = src_args
    else:
        sew, dsth, dinv, znd = src_args
    c = lax.axis_index("c")
    s = lax.axis_index("s")
    wid = s * NC + c
    base_n = s * NPT
    pltpu.sync_copy(znd.at[pl.ds(base_n, NPT)], acc.at[pl.ds(base_n, NPT)])
    if mode == 0:
        z16 = jnp.zeros((L,), jnp.float32)

        def zz(i, cc):
            tblv[pl.ds(i * L, L)] = z16
            return cc
        lax.fori_loop(0, NP // L, zz, 0)
    else:
        pltpu.sync_copy(dinv, tblv)
    plsc.subcore_barrier()

    pbase = s * (na + nb) + c * na
    nmine = jnp.where(c == 0, na, nb)

    def phase(setidx, half, rows, sem, sems):
        pltpu.make_async_copy(hlin.at[seb.at[setidx, pl.ds(0, B)]],
                              rows, sem).wait()
        sb = half * 2 * B
        wb = sb + B
        for grp in range(B // L):
            w16 = plsc.bitcast(seb[setidx, pl.ds(wb + grp * L, L)],
                               jnp.float32)
            d16 = dstb[setidx, half, pl.ds(grp * L, L)]
            if mode == 0:
                plsc.addupdate_scatter(tblv, [d16], w16)
                cv = w16
            else:
                s16 = seb[setidx, pl.ds(sb + grp * L, L)]
                cv = plsc.load_gather(tblv, [s16]) * w16 \
                    * plsc.load_gather(tblv, [d16])
            coefv[pl.ds(grp * L, L)] = cv
        _scale_rows(rows, coefv)
        pltpu.async_copy(rows, acc.at[dstb.at[setidx, half]], sems, add=True)

    def wait_scatter(rows, sems):
        pltpu.make_async_copy(rows, acc.at[dstb.at[0, 0]], sems).wait()

    def issue(setidx, half, rows, sem):
        pltpu.async_copy(hlin.at[seb.at[setidx, pl.ds(half * 2 * B, B)]],
                         rows, sem)

    _load_set(sew, dsth, seb, dstb, 0, pbase)
    issue(0, 0, rowsa, sema)
    issue(0, 1, rowsb, semb)
    _load_set(sew, dsth, seb, dstb, 1, pbase + 1)

    def body(g, carry):
        setidx = lax.rem(g, 2)
        nset = 1 - setidx
        phase(setidx, 0, rowsa, sema, semsa)
        phase(setidx, 1, rowsb, semb, semsb)
        wait_scatter(rowsa, semsa)
        issue(nset, 0, rowsa, sema)
        wait_scatter(rowsb, semsb)
        issue(nset, 1, rowsb, semb)
        _load_set(sew, dsth, seb, dstb, setidx, pbase + g + 2)
        return carry

    lax.fori_loop(0, nmine, body, 0)
    pltpu.make_async_copy(hlin.at[seb.at[0, pl.ds(0, B)]], rowsa, sema).wait()
    pltpu.make_async_copy(hlin.at[seb.at[0, pl.ds(0, B)]], rowsb, semb).wait()

    plsc.subcore_barrier()
    pltpu.sync_copy(acc.at[pl.ds(base_n, NPT)], mp.at[c, pl.ds(base_n, NPT)])
    if mode == 0:
        pltpu.sync_copy(tblv, dp_or_none.at[pl.ds(wid * NP, NP)])


def _edge0_body(na, nb, hlin, sew, dsth, znd, mp, dp, *scratch):
    _edge_pass_body(na, nb, 0, hlin, (sew, dsth, znd), mp, dp, *scratch)


def _edge_norm_body(na, nb, hlin, sew, dsth, dinv, znd, mp, *scratch):
    _edge_pass_body(na, nb, 1, hlin, (sew, dsth, dinv, znd), mp, None, *scratch)


_SC_SCRATCH = [
    pltpu.VMEM((2, 4 * B), jnp.int32),
    pltpu.VMEM((2, 2, B), jnp.int32),
    pltpu.VMEM((B,), jnp.float32),
    pltpu.VMEM((B, D), jnp.float32),
    pltpu.VMEM((B, D), jnp.float32),
    pltpu.VMEM((NP,), jnp.float32),
    pltpu.VMEM_SHARED((NP, D), jnp.float32),
    pltpu.SemaphoreType.DMA,
    pltpu.SemaphoreType.DMA,
    pltpu.SemaphoreType.DMA,
    pltpu.SemaphoreType.DMA,
]


def _edge0_call(na, nb, hlin, sew, dsth, znd):
    return pl.kernel(
        functools.partial(_edge0_body, na, nb),
        out_type=[jax.ShapeDtypeStruct((NC, NP, D), jnp.float32),
                  jax.ShapeDtypeStruct((NW * NP,), jnp.float32)],
        mesh=_MESH,
        compiler_params=_SC_PARAMS,
        scratch_types=_SC_SCRATCH,
    )(hlin, sew, dsth, znd)


def _edge_norm_call(na, nb, hlin, sew, dsth, dinv, znd):
    return pl.kernel(
        functools.partial(_edge_norm_body, na, nb),
        out_type=jax.ShapeDtypeStruct((NC, NP, D), jnp.float32),
        mesh=_MESH,
        compiler_params=_SC_PARAMS,
        scratch_types=_SC_SCRATCH,
    )(hlin, sew, dsth, dinv, znd)


def _row_spec(shape_tail):
    nt = len(shape_tail)
    return pl.BlockSpec((RB,) + shape_tail, lambda i, _nt=nt: (i,) + (0,) * _nt)


def _full_spec(shape):
    nd = len(shape)
    return pl.BlockSpec(shape, lambda i, _nd=nd: (0,) * _nd)


def _parts_spec():
    return pl.BlockSpec((NC, RB, D), lambda i: (0, i, 0))


def _tc_mm_body(x, w, o):
    o[...] = jnp.dot(x[...], w[...], preferred_element_type=jnp.float32,
                     precision=_HIGH)


def _tc_deg_body(dp, dinv_o):
    deg = jnp.sum(dp[...], axis=0) + 1.0
    dinv_o[...] = lax.rsqrt(deg)


def _tc_comb_body(mp, b, w, h_o, hlin_o):
    m = mp[0] + mp[1] + b[...][None, :]
    h = _leaky(m)
    h_o[...] = h
    hlin_o[...] = jnp.dot(h, w[...], preferred_element_type=jnp.float32,
                          precision=_HIGH)


def _tc_final_body(mp, b, h1, h2, wh, bh, o):
    m = mp[0] + mp[1] + b[...][None, :]
    h3 = _leaky(m)
    acc = jnp.dot(h1[...], wh[0:D, :], preferred_element_type=jnp.float32,
                  precision=_HIGH)
    acc += jnp.dot(h2[...], wh[D:2 * D, :], preferred_element_type=jnp.float32,
                   precision=_HIGH)
    acc += jnp.dot(h3, wh[2 * D:3 * D, :], preferred_element_type=jnp.float32,
                   precision=_HIGH)
    o[...] = acc + bh[...][None, :]


_GRID = NP // RB
_F32 = jnp.float32


def _sds(shape):
    return jax.ShapeDtypeStruct(shape, _F32)


def _pack_edges(src, dst, ew, totpairs):
    tot = NS * totpairs * 2 * B
    pad = tot - src.shape[0]
    z = jnp.zeros((pad,), jnp.int32)
    srcp = jnp.concatenate([src, z]).reshape(NS, totpairs, 2, B)
    dstp = jnp.concatenate([dst, z]).reshape(NS, totpairs, 2, B)
    ewb = lax.bitcast_convert_type(
        jnp.concatenate([ew, jnp.zeros((pad,), jnp.float32)]),
        jnp.int32).reshape(NS, totpairs, 2, B)
    sew = jnp.stack([srcp[:, :, 0], ewb[:, :, 0],
                     srcp[:, :, 1], ewb[:, :, 1]], axis=2)
    zp4 = jnp.zeros((2 * 4 * B,), jnp.int32)
    zp2 = jnp.zeros((2 * 2 * B,), jnp.int32)
    sew = jnp.concatenate([sew.reshape(-1), zp4])
    dstf = jnp.concatenate([dstp.reshape(-1), zp2])
    return sew, dstf


def kernel(x, edge_index, edge_weight, W0, b0, W1, b1, W2, b2, Wh, bh):
    E = edge_index.shape[1]
    src = edge_index[0].astype(jnp.int32)
    dst = edge_index[1].astype(jnp.int32)
    ew = edge_weight.astype(jnp.float32)

    tot0 = -(-E // (NS * 2 * B))
    na0 = _SPLIT0_NUM * tot0 // _SPLIT0_DEN
    nb0 = tot0 - na0
    sew0, dst0 = _pack_edges(src, dst, ew, tot0)

    loop = jnp.arange(N, dtype=jnp.int32)
    tot1 = -(-(E + N) // (NS * 2 * B))
    na1 = _SPLIT_NUM * tot1 // _SPLIT_DEN
    nb1 = tot1 - na1
    sew1, dst1 = _pack_edges(jnp.concatenate([src, loop]),
                             jnp.concatenate([dst, loop]),
                             jnp.concatenate([ew, jnp.ones((N,), jnp.float32)]),
                             tot1)

    xp = jnp.concatenate([x, jnp.zeros((NP - N, x.shape[1]), jnp.float32)])
    znd = jnp.zeros((NP, D), jnp.float32)

    mm = pl.pallas_call(
        _tc_mm_body, grid=(_GRID,),
        in_specs=[_row_spec((D,)), _full_spec((D, D))],
        out_specs=_row_spec((D,)),
        out_shape=_sds((NP, D)))

    hlin0 = mm(xp, W0)
    m0p, degp = _edge0_call(na0, nb0, hlin0, sew0, dst0, znd)

    dinv = pl.pallas_call(
        _tc_deg_body, grid=(1,),
        in_specs=[pl.BlockSpec((NW, NP), lambda i: (0, 0))],
        out_specs=_full_spec((NP,)),
        out_shape=_sds((NP,)))(degp.reshape(NW, NP))

    comb = pl.pallas_call(
        _tc_comb_body, grid=(_GRID,),
        in_specs=[_parts_spec(), _full_spec((D,)), _full_spec((D, D))],
        out_specs=[_row_spec((D,)), _row_spec((D,))],
        out_shape=[_sds((NP, D)), _sds((NP, D))])

    h1, hlin1 = comb(m0p, b0, W1)
    m1p = _edge_norm_call(na1, nb1, hlin1, sew1, dst1, dinv, znd)
    h2, hlin2 = comb(m1p, b1, W2)
    m2p = _edge_norm_call(na1, nb1, hlin2, sew1, dst1, dinv, znd)

    out = pl.pallas_call(
        _tc_final_body, grid=(_GRID,),
        in_specs=[_parts_spec(), _full_spec((D,)),
                  _row_spec((D,)), _row_spec((D,)),
                  _full_spec((3 * D, D)), _full_spec((D,))],
        out_specs=_row_spec((D,)),
        out_shape=_sds((NP, D)),
    )(m2p, b2, h1, h2, Wh, bh)
    return out[0:N]


# speedup vs baseline: 1.0014x; 1.0014x over previous
import functools

import jax
import jax.numpy as jnp
from jax import lax
from jax.experimental import pallas as pl
from jax.experimental.pallas import tpu as pltpu
from jax.experimental.pallas import tpu_sc as plsc

N = 10000
D = 128
NEG_SLOP = 0.2

NC = 2
NS = 16
L = 16
NW = NC * NS
B = 128
NPT = 640
NP = NS * NPT
RB = 1024

_MESH = plsc.VectorSubcoreMesh(
    core_axis_name="c", subcore_axis_name="s", num_cores=NC, num_subcores=NS)
_SC_PARAMS = pltpu.CompilerParams(needs_layout_passes=False)
_SPLIT_NUM, _SPLIT_DEN = 21, 32
_SPLIT0_NUM, _SPLIT0_DEN = 23, 32
_HIGH = lax.Precision.HIGHEST


def _leaky(v):
    return jnp.where(v >= 0, v, NEG_SLOP * v)


def _scale_rows(rows, coefv):
    @plsc.parallel_loop(0, B, 1, unroll=4)
    def _(i):
        cvec = plsc.load_gather(coefv, [jnp.full((L,), i, jnp.int32)])
        for j in range(D // L):
            rows[i, pl.ds(j * L, L)] = rows[i, pl.ds(j * L, L)] * cvec


def _load_set(sew, dsth, seb, dstb, setidx, p):
    o4 = pl.multiple_of(p * (4 * B), B)
    o2 = pl.multiple_of(p * (2 * B), B)
    pltpu.sync_copy(sew.at[pl.ds(o4, 4 * B)], seb.at[setidx])
    pltpu.sync_copy(dsth.at[pl.ds(o2, B)], dstb.at[setidx, 0])
    pltpu.sync_copy(dsth.at[pl.ds(o2 + B, B)], dstb.at[setidx, 1])


def _edge_pass_body(na, nb, mode, hlin, src_args, mp, dp_or_none,
                    seb, dstb, coefv, rowsa, rowsb, tblv, acc,
                    sema, semb, semsa, semsb):
    if mode == 0:
        sew, dsth, znd = src_args
    else:
        sew, dsth, dinv, znd = src_args
    c = lax.axis_index("c")
    s = lax.axis_index("s")
    wid = s * NC + c
    base_n = s * NPT
    pltpu.sync_copy(znd.at[pl.ds(base_n, NPT)], acc.at[pl.ds(base_n, NPT)])
    if mode == 0:
        z16 = jnp.zeros((L,), jnp.float32)

        def zz(i, cc):
            tblv[pl.ds(i * L, L)] = z16
            return cc
        lax.fori_loop(0, NP // L, zz, 0)
    else:
        pltpu.sync_copy(dinv, tblv)
    plsc.subcore_barrier()

    pbase = s * (na + nb) + c * na
    nmine = jnp.where(c == 0, na, nb)

    def phase(setidx, half, rows, sem, sems):
        pltpu.make_async_copy(hlin.at[seb.at[setidx, pl.ds(0, B)]],
                              rows, sem).wait()
        sb = half * 2 * B
        wb = sb + B
        for grp in range(B // L):
            w16 = plsc.bitcast(seb[setidx, pl.ds(wb + grp * L, L)],
                               jnp.float32)
            d16 = dstb[setidx, half, pl.ds(grp * L, L)]
            if mode == 0:
                plsc.addupdate_scatter(tblv, [d16], w16)
                cv = w16
            else:
                s16 = seb[setidx, pl.ds(sb + grp * L, L)]
                cv = plsc.load_gather(tblv, [s16]) * w16 \
                    * plsc.load_gather(tblv, [d16])
            coefv[pl.ds(grp * L, L)] = cv
        _scale_rows(rows, coefv)
        pltpu.async_copy(rows, acc.at[dstb.at[setidx, half]], sems, add=True)

    def wait_scatter(rows, sems):
        pltpu.make_async_copy(rows, acc.at[dstb.at[0, 0]], sems).wait()

    def issue(setidx, half, rows, sem):
        pltpu.async_copy(hlin.at[seb.at[setidx, pl.ds(half * 2 * B, B)]],
                         rows, sem)

    _load_set(sew, dsth, seb, dstb, 0, pbase)
    issue(0, 0, rowsa, sema)
    issue(0, 1, rowsb, semb)
    _load_set(sew, dsth, seb, dstb, 1, pbase + 1)

    def body(g, carry):
        setidx = lax.rem(g, 2)
        nset = 1 - setidx
        phase(setidx, 0, rowsa, sema, semsa)
        phase(setidx, 1, rowsb, semb, semsb)
        wait_scatter(rowsa, semsa)
        issue(nset, 0, rowsa, sema)
        wait_scatter(rowsb, semsb)
        issue(nset, 1, rowsb, semb)
        _load_set(sew, dsth, seb, dstb, setidx, pbase + g + 2)
        return carry

    lax.fori_loop(0, nmine, body, 0)
    pltpu.make_async_copy(hlin.at[seb.at[0, pl.ds(0, B)]], rowsa, sema).wait()
    pltpu.make_async_copy(hlin.at[seb.at[0, pl.ds(0, B)]], rowsb, semb).wait()

    plsc.subcore_barrier()
    pltpu.sync_copy(acc.at[pl.ds(base_n, NPT)], mp.at[c, pl.ds(base_n, NPT)])
    if mode == 0:
        pltpu.sync_copy(tblv, dp_or_none.at[pl.ds(wid * NP, NP)])


def _edge0_body(na, nb, hlin, sew, dsth, znd, mp, dp, *scratch):
    _edge_pass_body(na, nb, 0, hlin, (sew, dsth, znd), mp, dp, *scratch)


def _edge_norm_body(na, nb, hlin, sew, dsth, dinv, znd, mp, *scratch):
    _edge_pass_body(na, nb, 1, hlin, (sew, dsth, dinv, znd), mp, None, *scratch)


_SC_SCRATCH = [
    pltpu.VMEM((2, 4 * B), jnp.int32),
    pltpu.VMEM((2, 2, B), jnp.int32),
    pltpu.VMEM((B,), jnp.float32),
    pltpu.VMEM((B, D), jnp.float32),
    pltpu.VMEM((B, D), jnp.float32),
    pltpu.VMEM((NP,), jnp.float32),
    pltpu.VMEM_SHARED((NP, D), jnp.float32),
    pltpu.SemaphoreType.DMA,
    pltpu.SemaphoreType.DMA,
    pltpu.SemaphoreType.DMA,
    pltpu.SemaphoreType.DMA,
]


def _edge0_call(na, nb, hlin, sew, dsth, znd):
    return pl.kernel(
        functools.partial(_edge0_body, na, nb),
        out_type=[jax.ShapeDtypeStruct((NC, NP, D), jnp.float32),
                  jax.ShapeDtypeStruct((NW * NP,), jnp.float32)],
        mesh=_MESH,
        compiler_params=_SC_PARAMS,
        scratch_types=_SC_SCRATCH,
    )(hlin, sew, dsth, znd)


def _edge_norm_call(na, nb, hlin, sew, dsth, dinv, znd):
    return pl.kernel(
        functools.partial(_edge_norm_body, na, nb),
        out_type=jax.ShapeDtypeStruct((NC, NP, D), jnp.float32),
        mesh=_MESH,
        compiler_params=_SC_PARAMS,
        scratch_types=_SC_SCRATCH,
    )(hlin, sew, dsth, dinv, znd)


def _row_spec(shape_tail):
    nt = len(shape_tail)
    return pl.BlockSpec((RB,) + shape_tail, lambda i, _nt=nt: (i,) + (0,) * _nt)


def _full_spec(shape):
    nd = len(shape)
    return pl.BlockSpec(shape, lambda i, _nd=nd: (0,) * _nd)


def _parts_spec():
    return pl.BlockSpec((NC, RB, D), lambda i: (0, i, 0))


def _tc_mm_body(x, w, o):
    o[...] = jnp.dot(x[...], w[...], preferred_element_type=jnp.float32,
                     precision=_HIGH)


def _tc_deg_body(dp, dinv_o):
    deg = jnp.sum(dp[...], axis=0) + 1.0
    dinv_o[...] = lax.rsqrt(deg)


def _tc_comb_body(mp, b, w, h_o, hlin_o):
    m = mp[0] + mp[1] + b[...][None, :]
    h = _leaky(m)
    h_o[...] = h
    hlin_o[...] = jnp.dot(h, w[...], preferred_element_type=jnp.float32,
                          precision=_HIGH)


def _tc_final_body(mp, b, h1, h2, wh, bh, o):
    m = mp[0] + mp[1] + b[...][None, :]
    h3 = _leaky(m)
    acc = jnp.dot(h1[...], wh[0:D, :], preferred_element_type=jnp.float32,
                  precision=_HIGH)
    acc += jnp.dot(h2[...], wh[D:2 * D, :], preferred_element_type=jnp.float32,
                   precision=_HIGH)
    acc += jnp.dot(h3, wh[2 * D:3 * D, :], preferred_element_type=jnp.float32,
                   precision=_HIGH)
    o[...] = acc + bh[...][None, :]


_GRID = NP // RB
_F32 = jnp.float32


def _sds(shape):
    return jax.ShapeDtypeStruct(shape, _F32)


def _pack_edges(src, dst, ew, totpairs):
    tot = NS * totpairs * 2 * B
    pad = tot - src.shape[0]
    z = jnp.zeros((pad,), jnp.int32)
    srcp = jnp.concatenate([src, z]).reshape(NS, totpairs, 2, B)
    dstp = jnp.concatenate([dst, z]).reshape(NS, totpairs, 2, B)
    ewb = lax.bitcast_convert_type(
        jnp.concatenate([ew, jnp.zeros((pad,), jnp.float32)]),
        jnp.int32).reshape(NS, totpairs, 2, B)
    sew = jnp.stack([srcp[:, :, 0], ewb[:, :, 0],
                     srcp[:, :, 1], ewb[:, :, 1]], axis=2)
    zp4 = jnp.zeros((2 * 4 * B,), jnp.int32)
    zp2 = jnp.zeros((2 * 2 * B,), jnp.int32)
    sew = jnp.concatenate([sew.reshape(-1), zp4])
    dstf = jnp.concatenate([dstp.reshape(-1), zp2])
    return sew, dstf


def kernel(x, edge_index, edge_weight, W0, b0, W1, b1, W2, b2, Wh, bh):
    E = edge_index.shape[1]
    src = edge_index[0].astype(jnp.int32)
    dst = edge_index[1].astype(jnp.int32)
    ew = edge_weight.astype(jnp.float32)

    tot0 = -(-E // (NS * 2 * B))
    na0 = _SPLIT0_NUM * tot0 // _SPLIT0_DEN
    nb0 = tot0 - na0
    sew0, dst0 = _pack_edges(src, dst, ew, tot0)

    loop = jnp.arange(N, dtype=jnp.int32)
    tot1 = -(-(E + N) // (NS * 2 * B))
    na1 = _SPLIT_NUM * tot1 // _SPLIT_DEN
    nb1 = tot1 - na1
    sew1, dst1 = _pack_edges(jnp.concatenate([src, loop]),
                             jnp.concatenate([dst, loop]),
                             jnp.concatenate([ew, jnp.ones((N,), jnp.float32)]),
                             tot1)

    xp = jnp.concatenate([x, jnp.zeros((NP - N, x.shape[1]), jnp.float32)])
    znd = jnp.zeros((NP, D), jnp.float32)

    mm = pl.pallas_call(
        _tc_mm_body, grid=(_GRID,),
        in_specs=[_row_spec((D,)), _full_spec((D, D))],
        out_specs=_row_spec((D,)),
        out_shape=_sds((NP, D)))

    hlin0 = mm(xp, W0)
    m0p, degp = _edge0_call(na0, nb0, hlin0, sew0, dst0, znd)

    dinv = pl.pallas_call(
        _tc_deg_body, grid=(1,),
        in_specs=[pl.BlockSpec((NW, NP), lambda i: (0, 0))],
        out_specs=_full_spec((NP,)),
        out_shape=_sds((NP,)))(degp.reshape(NW, NP))

    comb = pl.pallas_call(
        _tc_comb_body, grid=(_GRID,),
        in_specs=[_parts_spec(), _full_spec((D,)), _full_spec((D, D))],
        out_specs=[_row_spec((D,)), _row_spec((D,))],
        out_shape=[_sds((NP, D)), _sds((NP, D))])

    h1, hlin1 = comb(m0p, b0, W1)
    m1p = _edge_norm_call(na1, nb1, hlin1, sew1, dst1, dinv, znd)
    h2, hlin2 = comb(m1p, b1, W2)
    m2p = _edge_norm_call(na1, nb1, hlin2, sew1, dst1, dinv, znd)

    out = pl.pallas_call(
        _tc_final_body, grid=(_GRID,),
        in_specs=[_parts_spec(), _full_spec((D,)),
                  _row_spec((D,)), _row_spec((D,)),
                  _full_spec((3 * D, D)), _full_spec((D,))],
        out_specs=_row_spec((D,)),
        out_shape=_sds((NP, D)),
    )(m2p, b2, h1, h2, Wh, bh)
    return out[0:N]
